# baseline (speedup 1.0000x reference)
import dataclasses
import functools

import jax
import jax.numpy as jnp
from jax import lax
from jax.experimental import pallas as pl
from jax.experimental.pallas import tpu as pltpu
from jax.experimental.pallas import tpu_sc as plsc

NC = 2
NS = 16
NT = NC * NS
LANES = 16
PT = 128
NSUB = PT // LANES
NG = 4
NBUF = 8
PREF = 5


def _sc_mesh():
  return plsc.VectorSubcoreMesh(
      core_axis_name="c", subcore_axis_name="s", num_cores=NC, num_subcores=NS
  )


def _sc_params():
  cp = pltpu.CompilerParams()
  if "needs_layout_passes" in pltpu.CompilerParams.__dataclass_fields__:
    cp = dataclasses.replace(cp, needs_layout_passes=False)
  return cp


def _stage1(denomT, nomT, denom_tail, nom_tail, w):
  L, D, P = denomT.shape
  T = w.shape[0]
  GL = L // NG
  NFULL = P // PT
  TAILP = P - NFULL * PT
  BPT_LO = NFULL // NT
  BPT_HI = -(-NFULL // NT)
  MAXP = BPT_HI * PT + (TAILP and PT)

  @functools.partial(
      pl.kernel,
      out_type=jax.ShapeDtypeStruct((P,), jnp.float32),
      mesh=_sc_mesh(),
      scratch_types=[
          pltpu.VMEM((T,), jnp.float32),
          pltpu.VMEM((T,), jnp.float32),
          pltpu.VMEM((NBUF, D, PT), jnp.int32),
          pltpu.VMEM((2, L, PT), jnp.int32),
          pltpu.VMEM((NG * MAXP,), jnp.float32),
          pltpu.VMEM((MAXP,), jnp.float32),
          pltpu.VMEM((max(TAILP * L * D, 1),), jnp.int32),
          pltpu.VMEM((max(TAILP * L, 1),), jnp.int32),
          pltpu.SemaphoreType.DMA((NBUF,)),
          pltpu.SemaphoreType.DMA((2,)),
      ],
      compiler_params=_sc_params(),
  )
  def body(dT_hbm, nT_hbm, dtail_hbm, ntail_hbm, w_hbm, prob_hbm,
           twv, etwv, dblk, nblk, m8v, nsv, dtv, ntv, dsem, nsem):
    wid = lax.axis_index("s") * NC + lax.axis_index("c")
    pltpu.sync_copy(w_hbm, twv)
    for i in range(T // LANES):
      sl = pl.ds(i * LANES, LANES)
      etwv[sl] = jnp.exp(twv[sl])

    lo = (wid * NFULL) // NT
    hi = ((wid + 1) * NFULL) // NT

    def start_slab(pt_idx, l, slot):
      pltpu.async_copy(
          dT_hbm.at[l, :, pl.ds(pt_idx * PT, PT)], dblk.at[slot],
          dsem.at[slot])

    def wait_slab(slot):
      pltpu.make_async_copy(
          dT_hbm.at[0, :, pl.ds(0, PT)], dblk.at[slot],
          dsem.at[slot]).wait()

    def start_nom(pt_idx, slot):
      pltpu.async_copy(
          nT_hbm.at[:, pl.ds(pt_idx * PT, PT)], nblk.at[slot],
          nsem.at[slot])

    def wait_nom(slot):
      pltpu.make_async_copy(
          nT_hbm.at[:, pl.ds(0, PT)], nblk.at[slot], nsem.at[slot]).wait()

    for j in range(PREF):
      pl.when(lo + 0 < hi)(lambda j=j: start_slab(lo, j, j))
    pl.when(lo < hi)(lambda: start_nom(lo, 0))

    ones = jnp.ones((LANES,), jnp.float32)

    @pl.loop(lo, hi)
    def _(b):
      i = b - lo
      ou = i * PT
      nslot = i & 1
      wait_nom(nslot)
      pl.when(b + 1 < hi)(lambda: start_nom(b + 1, (i + 1) & 1))
      for g in range(NG):
        for sub in range(NSUB):
          m8v[pl.ds(g * MAXP + ou + sub * LANES, LANES)] = ones

      @pl.loop(0, L)
      def _(l):
        slot = l & (NBUF - 1)
        pslot = (l + PREF) & (NBUF - 1)
        wait_slab(slot)
        pl.when(l + PREF < L)(lambda: start_slab(b, l + PREF, pslot))
        pl.when((l + PREF >= L) & (b + 1 < hi))(
            lambda: start_slab(b + 1, l + PREF - L, pslot))
        goff = (l // GL) * MAXP + ou
        for sub in range(NSUB):
          acc = None
          for d in range(D):
            idx = dblk[slot, d, pl.ds(sub * LANES, LANES)]
            v = plsc.load_gather(etwv, [idx])
            acc = v if acc is None else acc + v
          msl = pl.ds(goff + sub * LANES, LANES)
          m8v[msl] = m8v[msl] * acc
      for sub in range(NSUB):
        accn = None
        for l in range(L):
          idx = nblk[nslot, l, pl.ds(sub * LANES, LANES)]
          v = plsc.load_gather(twv, [idx])
          accn = v if accn is None else accn + v
        nsv[pl.ds(ou + sub * LANES, LANES)] = accn

    iota = jnp.arange(LANES, dtype=jnp.int32)
    if TAILP:
      otail = BPT_HI * PT

      @pl.when(wid == 0)
      def _():
        pltpu.sync_copy(dtail_hbm, dtv)
        pltpu.sync_copy(ntail_hbm, ntv)
        iota_ld = iota * (L * D)
        iota_l = iota * L
        ones = jnp.ones((LANES,), jnp.float32)
        for tb in range(TAILP // LANES):
          pbase = tb * LANES
          for g in range(NG):
            m8v[pl.ds(g * MAXP + otail + pbase, LANES)] = ones

          @pl.loop(0, L)
          def _(l):
            acc = None
            for d in range(D):
              tidx = plsc.load_gather(
                  dtv, [iota_ld + (pbase * L * D + l * D + d)])
              v = plsc.load_gather(etwv, [tidx])
              acc = v if acc is None else acc + v
            msl = pl.ds((l // GL) * MAXP + otail + pbase, LANES)
            m8v[msl] = m8v[msl] * acc
          accn = None
          for l in range(L):
            tidx = plsc.load_gather(ntv, [iota_l + (pbase * L + l)])
            v = plsc.load_gather(twv, [tidx])
            accn = v if accn is None else accn + v
          nsv[pl.ds(otail + pbase, LANES)] = accn

    LOG2C = (-3.2352098285324664, 7.085102749634603, -7.396151552156677,
             5.673521559327593, -2.9144927004919374, 0.9507418392608009,
             -0.17810974419475303, 0.014598489293481237)
    LN2 = 0.6931471805599453

    def log2v(m):
      bits = plsc.bitcast(m, jnp.int32)
      e = (bits >> 23) - 127
      f = plsc.bitcast((bits & 0x7FFFFF) | 0x3F800000, jnp.float32)
      acc = jnp.full((LANES,), LOG2C[7], jnp.float32)
      for a in LOG2C[6::-1]:
        acc = acc * f + a
      return e.astype(jnp.float32) + acc

    def prob_sub(off):
      sl = pl.ds(off, LANES)
      s = None
      for g in range(NG):
        lg = log2v(m8v[pl.ds(g * MAXP + off, LANES)])
        s = lg if s is None else s + lg
      nsv[sl] = jnp.exp(nsv[sl] - s * LN2)

    @pl.loop(lo, hi)
    def _(b):
      off0 = (b - lo) * PT
      for sub in range(NSUB):
        prob_sub(off0 + sub * LANES)

    if TAILP:
      @pl.when(wid == 0)
      def _():
        for tb in range(TAILP // LANES):
          prob_sub(BPT_HI * PT + tb * LANES)

    plo = lo * PT

    def flush(npaths, src_off, dst_off):
      def go():
        pltpu.sync_copy(nsv.at[pl.ds(src_off, npaths)],
                        prob_hbm.at[pl.ds(dst_off, npaths)])
      return go

    nb = hi - lo
    pl.when(nb == BPT_HI)(flush(BPT_HI * PT, 0, plo))
    if BPT_LO != BPT_HI:
      pl.when(nb == BPT_LO)(flush(BPT_LO * PT, 0, plo))
    if TAILP:
      pl.when(wid == 0)(flush(TAILP, BPT_HI * PT, NFULL * PT))

  return body(denomT, nomT, denom_tail, nom_tail, w)


def _stage3(prob, v2pT, v2p_tail):
  P = prob.shape[0]
  K, V = v2pT.shape
  NFULL = V // PT
  TAILV = V - NFULL * PT

  @functools.partial(
      pl.kernel,
      out_type=jax.ShapeDtypeStruct((V,), jnp.float32),
      mesh=_sc_mesh(),
      scratch_types=[
          pltpu.VMEM((P,), jnp.float32),
          pltpu.VMEM((2, K, PT), jnp.int32),
          pltpu.VMEM((PT,), jnp.float32),
          pltpu.VMEM((max(TAILV * K, 1),), jnp.int32),
          pltpu.SemaphoreType.DMA((2,)),
      ],
      compiler_params=_sc_params(),
  )
  def body(prob_hbm, v2pT_hbm, vtail_hbm, out_hbm, probv, vblk, accv, vtv,
           sem):
    wid = lax.axis_index("s") * NC + lax.axis_index("c")
    pltpu.sync_copy(prob_hbm, probv)
    lo = (wid * NFULL) // NT
    hi = ((wid + 1) * NFULL) // NT

    def start_blk(vt, slot):
      pltpu.async_copy(
          v2pT_hbm.at[:, pl.ds(vt * PT, PT)], vblk.at[slot], sem.at[slot])

    def wait_blk(slot):
      pltpu.make_async_copy(
          v2pT_hbm.at[:, pl.ds(0, PT)], vblk.at[slot], sem.at[slot]).wait()

    pl.when(lo < hi)(lambda: start_blk(lo, 0))

    @pl.loop(lo, hi)
    def _(b):
      i = b - lo
      slot = i & 1
      wait_blk(slot)
      pl.when(b + 1 < hi)(lambda: start_blk(b + 1, (i + 1) & 1))
      for sub in range(NSUB):
        acc = None
        for k in range(K):
          idx = vblk[slot, k, pl.ds(sub * LANES, LANES)]
          v = plsc.load_gather(probv, [idx])
          acc = v if acc is None else acc + v
        accv[pl.ds(sub * LANES, LANES)] = acc
      pltpu.sync_copy(accv, out_hbm.at[pl.ds(b * PT, PT)])

    if TAILV:
      iota = jnp.arange(LANES, dtype=jnp.int32)
      iota_k = iota * K

      @pl.when(wid == 0)
      def _():
        pltpu.sync_copy(vtail_hbm, vtv)
        for tb in range(TAILV // LANES):
          acc = None
          for k in range(K):
            pidx = plsc.load_gather(vtv, [iota_k + (tb * LANES * K + k)])
            v = plsc.load_gather(probv, [pidx])
            acc = v if acc is None else acc + v
          accv[pl.ds(tb * LANES, LANES)] = acc
        pltpu.sync_copy(accv.at[pl.ds(0, TAILV)],
                        out_hbm.at[pl.ds(NFULL * PT, TAILV)])

  return body(prob, v2pT, v2p_tail)


def kernel(variant_2_paths, paths_nom, paths_denom, w_transitions):
  P, L, D = paths_denom.shape
  V, K = variant_2_paths.shape
  NFULL_P = P // PT
  NFULL_V = V // PT
  denomT = jnp.transpose(paths_denom, (1, 2, 0))
  nomT = jnp.transpose(paths_nom, (1, 0))
  v2pT = jnp.transpose(variant_2_paths, (1, 0))
  denom_tail = paths_denom[NFULL_P * PT:].reshape(-1)
  nom_tail = paths_nom[NFULL_P * PT:].reshape(-1)
  v2p_tail = variant_2_paths[NFULL_V * PT:].reshape(-1)
  prob = _stage1(denomT, nomT, denom_tail, nom_tail, w_transitions)
  return _stage3(prob, v2pT, v2p_tail)


# speedup vs baseline: 1.0061x; 1.0061x over previous
import dataclasses
import functools

import jax
import jax.numpy as jnp
from jax import lax
from jax.experimental import pallas as pl
from jax.experimental.pallas import tpu as pltpu
from jax.experimental.pallas import tpu_sc as plsc

NC = 2
NS = 16
NT = NC * NS
LANES = 16
PT = 128
NSUB = PT // LANES
NG = 4
NBUF = 8
PREF = 5


def _sc_mesh():
  return plsc.VectorSubcoreMesh(
      core_axis_name="c", subcore_axis_name="s", num_cores=NC, num_subcores=NS
  )


def _sc_params():
  cp = pltpu.CompilerParams()
  if "needs_layout_passes" in pltpu.CompilerParams.__dataclass_fields__:
    cp = dataclasses.replace(cp, needs_layout_passes=False)
  return cp


def _stage1(denomT, nomT, denom_tail, nom_tail, w):
  L, D, P = denomT.shape
  T = w.shape[0]
  GL = L // NG
  NFULL = P // PT
  TAILP = P - NFULL * PT
  BPT_LO = NFULL // NT
  BPT_HI = -(-NFULL // NT)
  MAXP = BPT_HI * PT + (TAILP and PT)

  @functools.partial(
      pl.kernel,
      out_type=tuple(
          [jax.ShapeDtypeStruct((P,), jnp.float32) for _ in range(NG)]
          + [jax.ShapeDtypeStruct((P,), jnp.float32)]
      ),
      mesh=_sc_mesh(),
      scratch_types=[
          pltpu.VMEM((T,), jnp.float32),
          pltpu.VMEM((T,), jnp.float32),
          pltpu.VMEM((NBUF, D, PT), jnp.int32),
          pltpu.VMEM((2, L, PT), jnp.int32),
          pltpu.VMEM((NG * MAXP,), jnp.float32),
          pltpu.VMEM((MAXP,), jnp.float32),
          pltpu.VMEM((max(TAILP * L * D, 1),), jnp.int32),
          pltpu.VMEM((max(TAILP * L, 1),), jnp.int32),
          pltpu.SemaphoreType.DMA((NBUF,)),
          pltpu.SemaphoreType.DMA((2,)),
      ],
      compiler_params=_sc_params(),
  )
  def body(dT_hbm, nT_hbm, dtail_hbm, ntail_hbm, w_hbm,
           m0_hbm, m1_hbm, m2_hbm, m3_hbm, ns_hbm,
           twv, etwv, dblk, nblk, m8v, nsv, dtv, ntv, dsem, nsem):
    wid = lax.axis_index("s") * NC + lax.axis_index("c")
    pltpu.sync_copy(w_hbm, twv)
    for i in range(T // LANES):
      sl = pl.ds(i * LANES, LANES)
      etwv[sl] = jnp.exp(twv[sl])

    lo = (wid * NFULL) // NT
    hi = ((wid + 1) * NFULL) // NT

    def start_slab(pt_idx, l, slot):
      pltpu.async_copy(
          dT_hbm.at[l, :, pl.ds(pt_idx * PT, PT)], dblk.at[slot],
          dsem.at[slot])

    def wait_slab(slot):
      pltpu.make_async_copy(
          dT_hbm.at[0, :, pl.ds(0, PT)], dblk.at[slot],
          dsem.at[slot]).wait()

    def start_nom(pt_idx, slot):
      pltpu.async_copy(
          nT_hbm.at[:, pl.ds(pt_idx * PT, PT)], nblk.at[slot],
          nsem.at[slot])

    def wait_nom(slot):
      pltpu.make_async_copy(
          nT_hbm.at[:, pl.ds(0, PT)], nblk.at[slot], nsem.at[slot]).wait()

    for j in range(PREF):
      pl.when(lo + 0 < hi)(lambda j=j: start_slab(lo, j, j))
    pl.when(lo < hi)(lambda: start_nom(lo, 0))

    ones = jnp.ones((LANES,), jnp.float32)

    @pl.loop(lo, hi)
    def _(b):
      i = b - lo
      ou = i * PT
      nslot = i & 1
      wait_nom(nslot)
      pl.when(b + 1 < hi)(lambda: start_nom(b + 1, (i + 1) & 1))
      for g in range(NG):
        for sub in range(NSUB):
          m8v[pl.ds(g * MAXP + ou + sub * LANES, LANES)] = ones

      @pl.loop(0, L)
      def _(l):
        slot = l & (NBUF - 1)
        pslot = (l + PREF) & (NBUF - 1)
        wait_slab(slot)
        pl.when(l + PREF < L)(lambda: start_slab(b, l + PREF, pslot))
        pl.when((l + PREF >= L) & (b + 1 < hi))(
            lambda: start_slab(b + 1, l + PREF - L, pslot))
        goff = (l // GL) * MAXP + ou
        for sub in range(NSUB):
          acc = None
          for d in range(D):
            idx = dblk[slot, d, pl.ds(sub * LANES, LANES)]
            v = plsc.load_gather(etwv, [idx])
            acc = v if acc is None else acc + v
          msl = pl.ds(goff + sub * LANES, LANES)
          m8v[msl] = m8v[msl] * acc
      for sub in range(NSUB):
        accn = None
        for l in range(L):
          idx = nblk[nslot, l, pl.ds(sub * LANES, LANES)]
          v = plsc.load_gather(twv, [idx])
          accn = v if accn is None else accn + v
        nsv[pl.ds(ou + sub * LANES, LANES)] = accn

    iota = jnp.arange(LANES, dtype=jnp.int32)
    if TAILP:
      otail = BPT_HI * PT

      @pl.when(wid == 0)
      def _():
        pltpu.sync_copy(dtail_hbm, dtv)
        pltpu.sync_copy(ntail_hbm, ntv)
        iota_ld = iota * (L * D)
        iota_l = iota * L
        ones = jnp.ones((LANES,), jnp.float32)
        for tb in range(TAILP // LANES):
          pbase = tb * LANES
          for g in range(NG):
            m8v[pl.ds(g * MAXP + otail + pbase, LANES)] = ones

          @pl.loop(0, L)
          def _(l):
            acc = None
            for d in range(D):
              tidx = plsc.load_gather(
                  dtv, [iota_ld + (pbase * L * D + l * D + d)])
              v = plsc.load_gather(etwv, [tidx])
              acc = v if acc is None else acc + v
            msl = pl.ds((l // GL) * MAXP + otail + pbase, LANES)
            m8v[msl] = m8v[msl] * acc
          accn = None
          for l in range(L):
            tidx = plsc.load_gather(ntv, [iota_l + (pbase * L + l)])
            v = plsc.load_gather(twv, [tidx])
            accn = v if accn is None else accn + v
          nsv[pl.ds(otail + pbase, LANES)] = accn

    plo = lo * PT
    m_hbms = [m0_hbm, m1_hbm, m2_hbm, m3_hbm]

    def flush(npaths, src_off, dst_off):
      def go():
        for g in range(NG):
          pltpu.sync_copy(
              m8v.at[pl.ds(g * MAXP + src_off, npaths)],
              m_hbms[g].at[pl.ds(dst_off, npaths)],
          )
        pltpu.sync_copy(nsv.at[pl.ds(src_off, npaths)],
                        ns_hbm.at[pl.ds(dst_off, npaths)])
      return go

    nb = hi - lo
    pl.when(nb == BPT_HI)(flush(BPT_HI * PT, 0, plo))
    if BPT_LO != BPT_HI:
      pl.when(nb == BPT_LO)(flush(BPT_LO * PT, 0, plo))
    if TAILP:
      pl.when(wid == 0)(flush(TAILP, BPT_HI * PT, NFULL * PT))

  return body(denomT, nomT, denom_tail, nom_tail, w)


def _stage2(m8s, ns):
  P = ns.shape[0]

  def body(m0_ref, m1_ref, m2_ref, m3_ref, ns_ref, prob_ref):
    s = (jnp.log(m0_ref[...]) + jnp.log(m1_ref[...])
         + jnp.log(m2_ref[...]) + jnp.log(m3_ref[...]))
    prob_ref[...] = jnp.exp(ns_ref[...] - s)

  return pl.pallas_call(
      body, out_shape=jax.ShapeDtypeStruct((P,), jnp.float32)
  )(*m8s, ns)


def _stage3(prob, v2pT, v2p_tail):
  P = prob.shape[0]
  K, V = v2pT.shape
  NFULL = V // PT
  TAILV = V - NFULL * PT

  @functools.partial(
      pl.kernel,
      out_type=jax.ShapeDtypeStruct((V,), jnp.float32),
      mesh=_sc_mesh(),
      scratch_types=[
          pltpu.VMEM((P,), jnp.float32),
          pltpu.VMEM((2, K, PT), jnp.int32),
          pltpu.VMEM((PT,), jnp.float32),
          pltpu.VMEM((max(TAILV * K, 1),), jnp.int32),
          pltpu.SemaphoreType.DMA((2,)),
      ],
      compiler_params=_sc_params(),
  )
  def body(prob_hbm, v2pT_hbm, vtail_hbm, out_hbm, probv, vblk, accv, vtv,
           sem):
    wid = lax.axis_index("s") * NC + lax.axis_index("c")
    pltpu.sync_copy(prob_hbm, probv)
    lo = (wid * NFULL) // NT
    hi = ((wid + 1) * NFULL) // NT

    def start_blk(vt, slot):
      pltpu.async_copy(
          v2pT_hbm.at[:, pl.ds(vt * PT, PT)], vblk.at[slot], sem.at[slot])

    def wait_blk(slot):
      pltpu.make_async_copy(
          v2pT_hbm.at[:, pl.ds(0, PT)], vblk.at[slot], sem.at[slot]).wait()

    pl.when(lo < hi)(lambda: start_blk(lo, 0))

    @pl.loop(lo, hi)
    def _(b):
      i = b - lo
      slot = i & 1
      wait_blk(slot)
      pl.when(b + 1 < hi)(lambda: start_blk(b + 1, (i + 1) & 1))
      for sub in range(NSUB):
        acc = None
        for k in range(K):
          idx = vblk[slot, k, pl.ds(sub * LANES, LANES)]
          v = plsc.load_gather(probv, [idx])
          acc = v if acc is None else acc + v
        accv[pl.ds(sub * LANES, LANES)] = acc
      pltpu.sync_copy(accv, out_hbm.at[pl.ds(b * PT, PT)])

    if TAILV:
      iota = jnp.arange(LANES, dtype=jnp.int32)
      iota_k = iota * K

      @pl.when(wid == 0)
      def _():
        pltpu.sync_copy(vtail_hbm, vtv)
        for tb in range(TAILV // LANES):
          acc = None
          for k in range(K):
            pidx = plsc.load_gather(vtv, [iota_k + (tb * LANES * K + k)])
            v = plsc.load_gather(probv, [pidx])
            acc = v if acc is None else acc + v
          accv[pl.ds(tb * LANES, LANES)] = acc
        pltpu.sync_copy(accv.at[pl.ds(0, TAILV)],
                        out_hbm.at[pl.ds(NFULL * PT, TAILV)])

  return body(prob, v2pT, v2p_tail)


def kernel(variant_2_paths, paths_nom, paths_denom, w_transitions):
  P, L, D = paths_denom.shape
  V, K = variant_2_paths.shape
  NFULL_P = P // PT
  NFULL_V = V // PT
  denomT = jnp.transpose(paths_denom, (1, 2, 0))
  nomT = jnp.transpose(paths_nom, (1, 0))
  v2pT = jnp.transpose(variant_2_paths, (1, 0))
  denom_tail = paths_denom[NFULL_P * PT:].reshape(-1)
  nom_tail = paths_nom[NFULL_P * PT:].reshape(-1)
  v2p_tail = variant_2_paths[NFULL_V * PT:].reshape(-1)
  *m8s, ns = _stage1(denomT, nomT, denom_tail, nom_tail, w_transitions)
  prob = _stage2(m8s, ns)
  return _stage3(prob, v2pT, v2p_tail)
